# Initial kernel scaffold; baseline (speedup 1.0000x reference)
import jax
import jax.numpy as jnp
from jax.experimental import pallas as pl


def kernel(coords, feats, gamma, beta, W, b):
    raise NotImplementedError("write your pallas kernel here")


# speedup vs baseline: 1.0463x; 1.0463x over previous
import functools

import jax
import jax.numpy as jnp
from jax import lax
from jax.experimental import pallas as pl
from jax.experimental.pallas import tpu as pltpu
from jax.experimental.pallas import tpu_sc as plsc

N = 16384
C_IN = 128
C_OUT = 256
K = 16
STRIDE = 4
M = N // STRIDE
EPS = 1e-5


_G_BLOCK = 512


def _g_body(feats_ref, gamma_ref, beta_ref, w_ref, b_ref, out_ref):
    x = feats_ref[...]
    mean = jnp.mean(x, axis=1, keepdims=True)
    xc = x - mean
    var = jnp.mean(xc * xc, axis=1, keepdims=True)
    normed = xc * lax.rsqrt(var + EPS) * gamma_ref[...] + beta_ref[...]
    out_ref[...] = (
        jnp.dot(normed, w_ref[...], preferred_element_type=jnp.float32)
        + b_ref[...]
    )


def _compute_g(feats, gamma, beta, w, b):
    return pl.pallas_call(
        _g_body,
        out_shape=jax.ShapeDtypeStruct((N, C_OUT), jnp.float32),
        grid=(N // _G_BLOCK,),
        in_specs=[
            pl.BlockSpec((_G_BLOCK, C_IN), lambda i: (i, 0)),
            pl.BlockSpec((1, C_IN), lambda i: (0, 0)),
            pl.BlockSpec((1, C_IN), lambda i: (0, 0)),
            pl.BlockSpec((C_IN, C_OUT), lambda i: (0, 0)),
            pl.BlockSpec((1, C_OUT), lambda i: (0, 0)),
        ],
        out_specs=pl.BlockSpec((_G_BLOCK, C_OUT), lambda i: (i, 0)),
    )(feats, gamma.reshape(1, C_IN), beta.reshape(1, C_IN), w, b.reshape(1, C_OUT))


_QCHUNK = 16
_NWORK = 32
_QPW = M // _NWORK
_LANES = 16


def _gather_max(g_hbm, idx_hbm, out_hbm, idx_v, rows_v, out_v, sem):
    wid = lax.axis_index("s") * 2 + lax.axis_index("c")

    def chunk_body(ci, carry):
        base_q = wid * _QPW + ci * _QCHUNK
        pltpu.sync_copy(idx_hbm.at[pl.ds(base_q * K, _QCHUNK * K)], idx_v)
        pltpu.async_copy(g_hbm.at[idx_v], rows_v, sem).wait()

        def q_body(q, c2):
            def col_body(cb, c3):
                col = pl.ds(cb * _LANES, _LANES)
                acc = rows_v[q * K, col]
                for k in range(1, K):
                    acc = jnp.maximum(acc, rows_v[q * K + k, col])
                out_v[q, col] = acc
                return c3

            return lax.fori_loop(0, C_OUT // _LANES, col_body, c2)

        lax.fori_loop(0, _QCHUNK, q_body, 0)
        pltpu.sync_copy(out_v, out_hbm.at[pl.ds(base_q, _QCHUNK)])
        return carry

    lax.fori_loop(0, _QPW // _QCHUNK, chunk_body, 0)


def _run_gather_max(g, idx_flat):
    mesh = plsc.VectorSubcoreMesh(core_axis_name="c", subcore_axis_name="s")
    fn = functools.partial(
        pl.kernel,
        mesh=mesh,
        out_type=jax.ShapeDtypeStruct((M, C_OUT), jnp.float32),
        scratch_types=[
            pltpu.VMEM((_QCHUNK * K,), jnp.int32),
            pltpu.VMEM((_QCHUNK * K, C_OUT), jnp.float32),
            pltpu.VMEM((_QCHUNK, C_OUT), jnp.float32),
            pltpu.SemaphoreType.DMA,
        ],
    )(_gather_max)
    return fn(g, idx_flat)


def _knn_idx(coords):
    coords_down = coords[:: STRIDE]
    d2 = (
        jnp.sum(coords_down * coords_down, axis=1, keepdims=True)
        - 2.0 * (coords_down @ coords.T)
        + jnp.sum(coords * coords, axis=1)[None, :]
    )
    _, knn_idx = lax.top_k(-d2, K)
    return knn_idx


def kernel(coords, feats, gamma, beta, W, b):
    g = _compute_g(feats, gamma, beta, W, b)
    knn_idx = _knn_idx(coords)
    return _run_gather_max(g, knn_idx.reshape(-1).astype(jnp.int32))


# speedup vs baseline: 4.0161x; 3.8385x over previous
import functools

import jax
import jax.numpy as jnp
from jax import lax
from jax.experimental import pallas as pl
from jax.experimental.pallas import tpu as pltpu
from jax.experimental.pallas import tpu_sc as plsc

N = 16384
C_IN = 128
C_OUT = 256
K = 16
STRIDE = 4
M = N // STRIDE
EPS = 1e-5


_G_BLOCK = 512


def _g_body(feats_ref, gamma_ref, beta_ref, w_ref, b_ref, out_ref):
    x = feats_ref[...]
    mean = jnp.mean(x, axis=1, keepdims=True)
    xc = x - mean
    var = jnp.mean(xc * xc, axis=1, keepdims=True)
    normed = xc * lax.rsqrt(var + EPS) * gamma_ref[...] + beta_ref[...]
    out_ref[...] = (
        jnp.dot(normed, w_ref[...], preferred_element_type=jnp.float32)
        + b_ref[...]
    )


def _compute_g(feats, gamma, beta, w, b):
    return pl.pallas_call(
        _g_body,
        out_shape=jax.ShapeDtypeStruct((N, C_OUT), jnp.float32),
        grid=(N // _G_BLOCK,),
        in_specs=[
            pl.BlockSpec((_G_BLOCK, C_IN), lambda i: (i, 0)),
            pl.BlockSpec((1, C_IN), lambda i: (0, 0)),
            pl.BlockSpec((1, C_IN), lambda i: (0, 0)),
            pl.BlockSpec((C_IN, C_OUT), lambda i: (0, 0)),
            pl.BlockSpec((1, C_OUT), lambda i: (0, 0)),
        ],
        out_specs=pl.BlockSpec((_G_BLOCK, C_OUT), lambda i: (i, 0)),
    )(feats, gamma.reshape(1, C_IN), beta.reshape(1, C_IN), w, b.reshape(1, C_OUT))


_QCHUNK = 16
_NWORK = 32
_QPW = M // _NWORK
_LANES = 16


def _gather_max(g_hbm, idx_hbm, out_hbm, idx_v, rows_v, out_v, sem):
    wid = lax.axis_index("s") * 2 + lax.axis_index("c")

    def chunk_body(ci, carry):
        base_q = wid * _QPW + ci * _QCHUNK
        pltpu.sync_copy(idx_hbm.at[pl.ds(base_q * K, _QCHUNK * K)], idx_v)
        pltpu.async_copy(g_hbm.at[idx_v], rows_v, sem).wait()

        def q_body(q, c2):
            def col_body(cb, c3):
                col = pl.ds(cb * _LANES, _LANES)
                acc = rows_v[q * K, col]
                for k in range(1, K):
                    acc = jnp.maximum(acc, rows_v[q * K + k, col])
                out_v[q, col] = acc
                return c3

            return lax.fori_loop(0, C_OUT // _LANES, col_body, c2)

        lax.fori_loop(0, _QCHUNK, q_body, 0)
        pltpu.sync_copy(out_v, out_hbm.at[pl.ds(base_q, _QCHUNK)])
        return carry

    lax.fori_loop(0, _QPW // _QCHUNK, chunk_body, 0)


def _run_gather_max(g, idx_flat):
    mesh = plsc.VectorSubcoreMesh(core_axis_name="c", subcore_axis_name="s")
    fn = functools.partial(
        pl.kernel,
        mesh=mesh,
        out_type=jax.ShapeDtypeStruct((M, C_OUT), jnp.float32),
        scratch_types=[
            pltpu.VMEM((_QCHUNK * K,), jnp.int32),
            pltpu.VMEM((_QCHUNK * K, C_OUT), jnp.float32),
            pltpu.VMEM((_QCHUNK, C_OUT), jnp.float32),
            pltpu.SemaphoreType.DMA,
        ],
    )(_gather_max)
    return fn(g, idx_flat)


_QB = 256


def _knn_body(q_ref, ct_ref, idx_ref):
    q = q_ref[...]
    ct = ct_ref[...]
    qn = jnp.sum(q * q, axis=1, keepdims=True)
    cn = jnp.sum(ct * ct, axis=0, keepdims=True)
    d = (qn - 2.0 * jnp.dot(q, ct, preferred_element_type=jnp.float32)) + cn
    iota = lax.broadcasted_iota(jnp.int32, (_QB, N), 1)
    for r in range(K):
        m = jnp.min(d, axis=1, keepdims=True)
        sel = jnp.where(d == m, iota, N)
        idx = jnp.min(sel, axis=1)
        idx_ref[:, r] = idx
        d = jnp.where(iota == idx[:, None], jnp.inf, d)


def _knn_idx(coords):
    coords_down = coords[::STRIDE]
    qpad = jnp.pad(coords_down, ((0, 0), (0, 5)))
    ctpad = jnp.pad(coords.T, ((0, 5), (0, 0)))
    return pl.pallas_call(
        _knn_body,
        out_shape=jax.ShapeDtypeStruct((M, K), jnp.int32),
        grid=(M // _QB,),
        in_specs=[
            pl.BlockSpec((_QB, 8), lambda i: (i, 0)),
            pl.BlockSpec((8, N), lambda i: (0, 0)),
        ],
        out_specs=pl.BlockSpec((_QB, K), lambda i: (i, 0)),
    )(qpad, ctpad)


def kernel(coords, feats, gamma, beta, W, b):
    g = _compute_g(feats, gamma, beta, W, b)
    knn_idx = _knn_idx(coords)
    return _run_gather_max(g, knn_idx.reshape(-1))
